# Initial kernel scaffold; baseline (speedup 1.0000x reference)
import jax
import jax.numpy as jnp
from jax.experimental import pallas as pl


def kernel(x, edge_index, batch, W1, b1, lin1_w, lin1_b, lin2_w, lin2_b):
    raise NotImplementedError("write your pallas kernel here")


# speedup vs baseline: 20.6545x; 20.6545x over previous
import functools

import jax
import jax.numpy as jnp
from jax import lax
from jax.experimental import pallas as pl
from jax.experimental.pallas import tpu as pltpu
from jax.experimental.pallas import tpu_sc as plsc

N = 10000
E = 320000
F = 128
G = 64

NC = 2
NS = 16
NW = NC * NS

EPT = E // NW
CH = 128
NCH = EPT // CH
TAIL = EPT - NCH * CH

SLC = 624
REM = N - NS * SLC
WCH = 104

BLK = 400
NBLK = N // BLK

_mesh = plsc.VectorSubcoreMesh(core_axis_name="c", subcore_axis_name="s")


@functools.partial(
    pl.kernel,
    out_type=jax.ShapeDtypeStruct((NC * N,), jnp.float32),
    mesh=_mesh,
    scratch_types=[
        pltpu.VMEM((CH,), jnp.float32),
        pltpu.VMEM((CH,), jnp.int32),
        pltpu.VMEM((TAIL,), jnp.int32),
        pltpu.VMEM((SLC,), jnp.float32),
        pltpu.VMEM_SHARED((N,), jnp.float32),
    ],
)
def _deg_sc(dst_hbm, deg_hbm, ones_v, idx_v, idx_t, zbuf, acc_sh):
    cid = lax.axis_index("c")
    sid = lax.axis_index("s")

    @pl.loop(0, SLC, step=16)
    def _(i):
        zbuf[pl.ds(i, 16)] = jnp.zeros((16,), jnp.float32)

    @pl.loop(0, CH, step=16)
    def _(i):
        ones_v[pl.ds(i, 16)] = jnp.ones((16,), jnp.float32)

    pltpu.sync_copy(zbuf, acc_sh.at[pl.ds(sid * SLC, SLC)])

    @pl.when(sid == NS - 1)
    def _():
        pltpu.sync_copy(zbuf.at[pl.ds(0, REM)], acc_sh.at[pl.ds(NS * SLC, REM)])

    plsc.subcore_barrier()

    base = (cid * NS + sid) * EPT

    @pl.loop(0, NCH * CH, step=CH)
    def _(j):
        pltpu.sync_copy(dst_hbm.at[pl.ds(base + j, CH)], idx_v)
        pltpu.sync_copy(ones_v, acc_sh.at[idx_v], add=True)

    pltpu.sync_copy(dst_hbm.at[pl.ds(base + NCH * CH, TAIL)], idx_t)
    pltpu.sync_copy(ones_v.at[pl.ds(0, TAIL)], acc_sh.at[idx_t], add=True)

    plsc.subcore_barrier()

    pltpu.sync_copy(acc_sh.at[pl.ds(sid * SLC, SLC)], zbuf)
    pltpu.sync_copy(zbuf, deg_hbm.at[pl.ds(cid * N + sid * SLC, SLC)])

    @pl.when(sid == NS - 1)
    def _():
        pltpu.sync_copy(acc_sh.at[pl.ds(NS * SLC, REM)], zbuf.at[pl.ds(0, REM)])
        pltpu.sync_copy(zbuf.at[pl.ds(0, REM)],
                        deg_hbm.at[pl.ds(cid * N + NS * SLC, REM)])


@functools.partial(
    pl.kernel,
    out_type=jax.ShapeDtypeStruct((NC * N, F), jnp.float32),
    mesh=_mesh,
    scratch_types=[
        pltpu.VMEM((CH,), jnp.int32),
        pltpu.VMEM((CH,), jnp.int32),
        pltpu.VMEM((TAIL,), jnp.int32),
        pltpu.VMEM((TAIL,), jnp.int32),
        pltpu.VMEM((CH, F), jnp.float32),
        pltpu.VMEM((TAIL, F), jnp.float32),
        pltpu.VMEM_SHARED((N, F), jnp.float32),
    ],
)
def _scatter_sc(xws_hbm, src_hbm, dst_hbm, acc_hbm,
                src_v, dst_v, src_t, dst_t, rows_v, rows_t, acc_sh):
    cid = lax.axis_index("c")
    sid = lax.axis_index("s")

    @pl.loop(0, SLC, step=WCH)
    def _(r):
        pltpu.sync_copy(xws_hbm.at[pl.ds(sid * SLC + r, WCH)],
                        rows_v.at[pl.ds(0, WCH)])
        pltpu.sync_copy(rows_v.at[pl.ds(0, WCH)],
                        acc_sh.at[pl.ds(sid * SLC + r, WCH)])

    @pl.when(sid == NS - 1)
    def _():
        pltpu.sync_copy(xws_hbm.at[pl.ds(NS * SLC, REM)], rows_t)
        pltpu.sync_copy(rows_t, acc_sh.at[pl.ds(NS * SLC, REM)])

    plsc.subcore_barrier()

    base = (cid * NS + sid) * EPT

    @pl.loop(0, NCH * CH, step=CH)
    def _(j):
        pltpu.sync_copy(src_hbm.at[pl.ds(base + j, CH)], src_v)
        pltpu.sync_copy(dst_hbm.at[pl.ds(base + j, CH)], dst_v)
        pltpu.sync_copy(xws_hbm.at[src_v], rows_v)
        pltpu.sync_copy(rows_v, acc_sh.at[dst_v], add=True)

    pltpu.sync_copy(src_hbm.at[pl.ds(base + NCH * CH, TAIL)], src_t)
    pltpu.sync_copy(dst_hbm.at[pl.ds(base + NCH * CH, TAIL)], dst_t)
    pltpu.sync_copy(xws_hbm.at[src_t], rows_t)
    pltpu.sync_copy(rows_t, acc_sh.at[dst_t], add=True)

    plsc.subcore_barrier()

    @pl.loop(0, SLC, step=WCH)
    def _(r):
        pltpu.sync_copy(acc_sh.at[pl.ds(sid * SLC + r, WCH)],
                        rows_v.at[pl.ds(0, WCH)])
        pltpu.sync_copy(rows_v.at[pl.ds(0, WCH)],
                        acc_hbm.at[pl.ds(cid * N + sid * SLC + r, WCH)])

    @pl.when(sid == NS - 1)
    def _():
        pltpu.sync_copy(acc_sh.at[pl.ds(NS * SLC, REM)], rows_t)
        pltpu.sync_copy(rows_t, acc_hbm.at[pl.ds(cid * N + NS * SLC, REM)])


def _scale_body(x_ref, w_ref, d0_ref, d1_ref, xws_ref, dis_ref):
    xw = jnp.dot(x_ref[...], w_ref[...], preferred_element_type=jnp.float32)
    deg = d0_ref[...] + d1_ref[...] + 1.0
    dis = lax.rsqrt(deg)
    xws_ref[...] = dis * xw
    dis_ref[...] = dis


def _scale_tc(x, W1, d0, d1):
    return pl.pallas_call(
        _scale_body,
        grid=(NBLK,),
        in_specs=[
            pl.BlockSpec((BLK, F), lambda i: (i, 0)),
            pl.BlockSpec((F, F), lambda i: (0, 0)),
            pl.BlockSpec((BLK, 1), lambda i: (i, 0)),
            pl.BlockSpec((BLK, 1), lambda i: (i, 0)),
        ],
        out_specs=[
            pl.BlockSpec((BLK, F), lambda i: (i, 0)),
            pl.BlockSpec((BLK, 1), lambda i: (i, 0)),
        ],
        out_shape=[
            jax.ShapeDtypeStruct((N, F), jnp.float32),
            jax.ShapeDtypeStruct((N, 1), jnp.float32),
        ],
    )(x, W1, d0, d1)


def _final_body(acc_ref, xws_ref, dis_ref, bt_ref, b1_ref,
                w1_ref, bb1_ref, w2_ref, bb2_ref, out_ref,
                pooled_ref, counts_ref):
    i = pl.program_id(0)

    @pl.when(i == 0)
    def _():
        pooled_ref[...] = jnp.zeros_like(pooled_ref)
        counts_ref[...] = jnp.zeros_like(counts_ref)

    a = acc_ref[0] + acc_ref[1] - xws_ref[...]
    h = jnp.maximum(a * dis_ref[...] + b1_ref[...], 0.0)
    gids = lax.broadcasted_iota(jnp.int32, (BLK, G), 1).astype(jnp.float32)
    p = (bt_ref[...] == gids).astype(jnp.float32)
    pooled_ref[...] += lax.dot_general(
        p, h, (((0,), (0,)), ((), ())), preferred_element_type=jnp.float32)
    counts_ref[...] += lax.dot_general(
        p, jnp.ones((BLK, 1), jnp.float32), (((0,), (0,)), ((), ())),
        preferred_element_type=jnp.float32)

    @pl.when(i == NBLK - 1)
    def _():
        pm = pooled_ref[...] / jnp.maximum(counts_ref[...], 1.0)
        z = jnp.maximum(
            jnp.dot(pm, w1_ref[...], preferred_element_type=jnp.float32)
            + bb1_ref[...], 0.0)
        out_ref[...] = (
            jnp.dot(z, w2_ref[...], preferred_element_type=jnp.float32)
            + bb2_ref[...])


def _final_tc(acc, xws, dis, batchf, b1, lin1_w, lin1_b, lin2_w, lin2_b):
    return pl.pallas_call(
        _final_body,
        grid=(NBLK,),
        in_specs=[
            pl.BlockSpec((NC, BLK, F), lambda i: (0, i, 0)),
            pl.BlockSpec((BLK, F), lambda i: (i, 0)),
            pl.BlockSpec((BLK, 1), lambda i: (i, 0)),
            pl.BlockSpec((BLK, 1), lambda i: (i, 0)),
            pl.BlockSpec((1, F), lambda i: (0, 0)),
            pl.BlockSpec((F, F), lambda i: (0, 0)),
            pl.BlockSpec((1, F), lambda i: (0, 0)),
            pl.BlockSpec((F, 1), lambda i: (0, 0)),
            pl.BlockSpec((1, 1), lambda i: (0, 0)),
        ],
        out_specs=pl.BlockSpec((G, 1), lambda i: (0, 0)),
        out_shape=jax.ShapeDtypeStruct((G, 1), jnp.float32),
        scratch_shapes=[
            pltpu.VMEM((G, F), jnp.float32),
            pltpu.VMEM((G, 1), jnp.float32),
        ],
    )(acc, xws, dis, batchf, b1, lin1_w, lin1_b, lin2_w, lin2_b)


def kernel(x, edge_index, batch, W1, b1, lin1_w, lin1_b, lin2_w, lin2_b):
    src = edge_index[0].astype(jnp.int32)
    dst = edge_index[1].astype(jnp.int32)

    deg_parts = _deg_sc(dst)
    d0 = deg_parts[:N].reshape(N, 1)
    d1 = deg_parts[N:].reshape(N, 1)

    xws, dis = _scale_tc(x, W1, d0, d1)

    acc = _scatter_sc(xws, src, dst).reshape(NC, N, F)

    batchf = batch.astype(jnp.float32).reshape(N, 1)
    return _final_tc(acc, xws, dis, batchf,
                     b1.reshape(1, F), lin1_w, lin1_b.reshape(1, F),
                     lin2_w, lin2_b.reshape(1, 1))


# speedup vs baseline: 71.4943x; 3.4614x over previous
import functools

import jax
import jax.numpy as jnp
from jax import lax
from jax.experimental import pallas as pl
from jax.experimental.pallas import tpu as pltpu
from jax.experimental.pallas import tpu_sc as plsc

N = 10000
E = 320000
F = 128
G = 64

NC = 2
NS = 16
NW = NC * NS

CH = 128
NJ = 80
GRP = 16
NG = NJ // GRP
EPT = NJ * CH
EPAD = NW * EPT
NPAD = 16

SLC = 624
REM = N - NS * SLC
WCH = 104

BLK = 400
NBLK = N // BLK

_mesh = plsc.VectorSubcoreMesh(core_axis_name="c", subcore_axis_name="s")


@functools.partial(
    pl.kernel,
    out_type=jax.ShapeDtypeStruct((NC * N,), jnp.float32),
    mesh=_mesh,
    scratch_types=[
        pltpu.VMEM((CH,), jnp.float32),
        pltpu.VMEM((NJ, CH), jnp.int32),
        pltpu.VMEM((SLC,), jnp.float32),
        pltpu.VMEM_SHARED((N + NPAD,), jnp.float32),
    ],
)
def _deg_sc(dst_hbm, deg_hbm, ones_v, idx_v, zbuf, acc_sh):
    cid = lax.axis_index("c")
    sid = lax.axis_index("s")
    wid = cid * NS + sid

    @pl.loop(0, SLC, step=16)
    def _(i):
        zbuf[pl.ds(i, 16)] = jnp.zeros((16,), jnp.float32)

    @pl.loop(0, CH, step=16)
    def _(i):
        ones_v[pl.ds(i, 16)] = jnp.ones((16,), jnp.float32)

    pltpu.sync_copy(dst_hbm.at[pl.ds(wid * NJ, NJ)], idx_v)
    pltpu.sync_copy(zbuf, acc_sh.at[pl.ds(sid * SLC, SLC)])

    @pl.when(sid == NS - 1)
    def _():
        pltpu.sync_copy(zbuf.at[pl.ds(0, REM + NPAD)],
                        acc_sh.at[pl.ds(NS * SLC, REM + NPAD)])

    plsc.subcore_barrier()

    @pl.loop(0, NJ)
    def _(j):
        pltpu.sync_copy(ones_v, acc_sh.at[idx_v.at[j]], add=True)

    plsc.subcore_barrier()

    pltpu.sync_copy(acc_sh.at[pl.ds(sid * SLC, SLC)], zbuf)
    pltpu.sync_copy(zbuf, deg_hbm.at[pl.ds(cid * N + sid * SLC, SLC)])

    @pl.when(sid == NS - 1)
    def _():
        pltpu.sync_copy(acc_sh.at[pl.ds(NS * SLC, REM)], zbuf.at[pl.ds(0, REM)])
        pltpu.sync_copy(zbuf.at[pl.ds(0, REM)],
                        deg_hbm.at[pl.ds(cid * N + NS * SLC, REM)])


@functools.partial(
    pl.kernel,
    out_type=jax.ShapeDtypeStruct((NC * N, F), jnp.float32),
    mesh=_mesh,
    scratch_types=[
        pltpu.VMEM((2 * GRP, CH), jnp.int32),
        pltpu.VMEM((CH, F), jnp.float32),
        pltpu.VMEM((CH, F), jnp.float32),
        pltpu.SemaphoreType.DMA,
        pltpu.SemaphoreType.DMA,
        pltpu.VMEM_SHARED((N + NPAD, F), jnp.float32),
    ],
)
def _scatter_sc(xws_hbm, ei_hbm, acc_hbm,
                ed_v, buf0, buf1, sem0, sem1, acc_sh):
    cid = lax.axis_index("c")
    sid = lax.axis_index("s")
    wid = cid * NS + sid

    @pl.loop(0, SLC, step=WCH)
    def _(r):
        pltpu.sync_copy(xws_hbm.at[pl.ds(sid * SLC + r, WCH)],
                        buf0.at[pl.ds(0, WCH)])
        pltpu.sync_copy(buf0.at[pl.ds(0, WCH)],
                        acc_sh.at[pl.ds(sid * SLC + r, WCH)])

    @pl.when(sid == NS - 1)
    def _():
        pltpu.sync_copy(xws_hbm.at[pl.ds(NS * SLC, REM)],
                        buf1.at[pl.ds(0, REM)])
        pltpu.sync_copy(buf1.at[pl.ds(0, REM)],
                        acc_sh.at[pl.ds(NS * SLC, REM)])

    plsc.subcore_barrier()

    @pl.loop(0, NG)
    def _(g):
        pltpu.sync_copy(
            ei_hbm.at[pl.ds(wid * (2 * NJ) + g * (2 * GRP), 2 * GRP)], ed_v)
        pltpu.make_async_copy(xws_hbm.at[ed_v.at[0]], buf0, sem0).start()
        pltpu.make_async_copy(xws_hbm.at[ed_v.at[2]], buf1, sem1).start()

        @pl.loop(0, GRP, step=2)
        def _(jj):
            pltpu.make_async_copy(xws_hbm.at[ed_v.at[2 * jj]], buf0,
                                  sem0).wait()
            pltpu.sync_copy(buf0, acc_sh.at[ed_v.at[2 * jj + 1]], add=True)

            @pl.when(jj + 2 < GRP)
            def _():
                pltpu.make_async_copy(xws_hbm.at[ed_v.at[2 * (jj + 2)]],
                                      buf0, sem0).start()

            pltpu.make_async_copy(xws_hbm.at[ed_v.at[2 * jj + 2]], buf1,
                                  sem1).wait()
            pltpu.sync_copy(buf1, acc_sh.at[ed_v.at[2 * jj + 3]], add=True)

            @pl.when(jj + 3 < GRP)
            def _():
                pltpu.make_async_copy(xws_hbm.at[ed_v.at[2 * (jj + 3)]],
                                      buf1, sem1).start()

    plsc.subcore_barrier()

    @pl.loop(0, SLC, step=WCH)
    def _(r):
        pltpu.sync_copy(acc_sh.at[pl.ds(sid * SLC + r, WCH)],
                        buf0.at[pl.ds(0, WCH)])
        pltpu.sync_copy(buf0.at[pl.ds(0, WCH)],
                        acc_hbm.at[pl.ds(cid * N + sid * SLC + r, WCH)])

    @pl.when(sid == NS - 1)
    def _():
        pltpu.sync_copy(acc_sh.at[pl.ds(NS * SLC, REM)],
                        buf1.at[pl.ds(0, REM)])
        pltpu.sync_copy(buf1.at[pl.ds(0, REM)],
                        acc_hbm.at[pl.ds(cid * N + NS * SLC, REM)])


def _scale_body(x_ref, w_ref, d0_ref, d1_ref, xws_ref, dis_ref):
    xw = jnp.dot(x_ref[...], w_ref[...], preferred_element_type=jnp.float32)
    deg = d0_ref[...] + d1_ref[...] + 1.0
    dis = lax.rsqrt(deg)
    xws_ref[...] = dis * xw
    dis_ref[...] = dis


def _scale_tc(x, W1, d0, d1):
    return pl.pallas_call(
        _scale_body,
        grid=(NBLK,),
        in_specs=[
            pl.BlockSpec((BLK, F), lambda i: (i, 0)),
            pl.BlockSpec((F, F), lambda i: (0, 0)),
            pl.BlockSpec((BLK, 1), lambda i: (i, 0)),
            pl.BlockSpec((BLK, 1), lambda i: (i, 0)),
        ],
        out_specs=[
            pl.BlockSpec((BLK, F), lambda i: (i, 0)),
            pl.BlockSpec((BLK, 1), lambda i: (i, 0)),
        ],
        out_shape=[
            jax.ShapeDtypeStruct((N, F), jnp.float32),
            jax.ShapeDtypeStruct((N, 1), jnp.float32),
        ],
    )(x, W1, d0, d1)


def _final_body(acc_ref, xws_ref, dis_ref, bt_ref, b1_ref,
                w1_ref, bb1_ref, w2_ref, bb2_ref, out_ref,
                pooled_ref, counts_ref):
    i = pl.program_id(0)

    @pl.when(i == 0)
    def _():
        pooled_ref[...] = jnp.zeros_like(pooled_ref)
        counts_ref[...] = jnp.zeros_like(counts_ref)

    a = acc_ref[0] + acc_ref[1] - xws_ref[...]
    h = jnp.maximum(a * dis_ref[...] + b1_ref[...], 0.0)
    gids = lax.broadcasted_iota(jnp.int32, (BLK, G), 1).astype(jnp.float32)
    p = (bt_ref[...] == gids).astype(jnp.float32)
    pooled_ref[...] += lax.dot_general(
        p, h, (((0,), (0,)), ((), ())), preferred_element_type=jnp.float32,
        precision=lax.Precision.HIGHEST)
    counts_ref[...] += lax.dot_general(
        p, jnp.ones((BLK, 1), jnp.float32), (((0,), (0,)), ((), ())),
        preferred_element_type=jnp.float32,
        precision=lax.Precision.HIGHEST)

    @pl.when(i == NBLK - 1)
    def _():
        pm = pooled_ref[...] / jnp.maximum(counts_ref[...], 1.0)
        z = jnp.maximum(
            jnp.dot(pm, w1_ref[...], preferred_element_type=jnp.float32)
            + bb1_ref[...], 0.0)
        out_ref[...] = (
            jnp.dot(z, w2_ref[...], preferred_element_type=jnp.float32)
            + bb2_ref[...])


def _final_tc(acc, xws, dis, batchf, b1, lin1_w, lin1_b, lin2_w, lin2_b):
    return pl.pallas_call(
        _final_body,
        grid=(NBLK,),
        in_specs=[
            pl.BlockSpec((NC, BLK, F), lambda i: (0, i, 0)),
            pl.BlockSpec((BLK, F), lambda i: (i, 0)),
            pl.BlockSpec((BLK, 1), lambda i: (i, 0)),
            pl.BlockSpec((BLK, 1), lambda i: (i, 0)),
            pl.BlockSpec((1, F), lambda i: (0, 0)),
            pl.BlockSpec((F, F), lambda i: (0, 0)),
            pl.BlockSpec((1, F), lambda i: (0, 0)),
            pl.BlockSpec((F, 1), lambda i: (0, 0)),
            pl.BlockSpec((1, 1), lambda i: (0, 0)),
        ],
        out_specs=pl.BlockSpec((G, 1), lambda i: (0, 0)),
        out_shape=jax.ShapeDtypeStruct((G, 1), jnp.float32),
        scratch_shapes=[
            pltpu.VMEM((G, F), jnp.float32),
            pltpu.VMEM((G, 1), jnp.float32),
        ],
    )(acc, xws, dis, batchf, b1, lin1_w, lin1_b, lin2_w, lin2_b)


def kernel(x, edge_index, batch, W1, b1, lin1_w, lin1_b, lin2_w, lin2_b):
    src = edge_index[0].astype(jnp.int32)
    dst = edge_index[1].astype(jnp.int32)

    npad = EPAD - E
    pad_i = jnp.arange(npad, dtype=jnp.int32)
    src_p = jnp.concatenate([src, pad_i % 512]).reshape(NW * NJ, CH)
    dst_p = jnp.concatenate([dst, N + (pad_i % NPAD)]).reshape(NW * NJ, CH)
    ei_p = jnp.stack([src_p, dst_p], axis=1).reshape(2 * NW * NJ, CH)

    deg_parts = _deg_sc(dst_p)
    d0 = deg_parts[:N].reshape(N, 1)
    d1 = deg_parts[N:].reshape(N, 1)

    xws, dis = _scale_tc(x, W1, d0, d1)

    acc = _scatter_sc(xws, ei_p).reshape(NC, N, F)

    batchf = batch.astype(jnp.float32).reshape(N, 1)
    return _final_tc(acc, xws, dis, batchf,
                     b1.reshape(1, F), lin1_w, lin1_b.reshape(1, F),
                     lin2_w, lin2_b.reshape(1, 1))
